# Initial kernel scaffold; baseline (speedup 1.0000x reference)
import jax
import jax.numpy as jnp
from jax.experimental import pallas as pl


def kernel(node_feats, edge_feats, edge_index, W1, b1, gamma, beta, W2, b2):
    raise NotImplementedError("write your pallas kernel here")


# speedup vs baseline: 1.8057x; 1.8057x over previous
import functools

import jax
import jax.numpy as jnp
from jax import lax
from jax.experimental import pallas as pl
from jax.experimental.pallas import tpu as pltpu
from jax.experimental.pallas import tpu_sc as plsc

N, E, D, DE = 10000, 320000, 128, 16
IDIM = 2 * D + DE

NC, NS = 2, 16
NW = NC * NS
E_PER = E // NW
CB = 80
ITERS = E_PER // CB

BE = 2000
NBLK = E // BE


def _gather_body(idx_i_hbm, idx_j_hbm, table_hbm, out_i_hbm, out_j_hbm,
                 idx_i_v, idx_j_v, rows_i, rows_j, sem_i, sem_j):
    wid = lax.axis_index("s") * NC + lax.axis_index("c")
    base0 = wid * E_PER

    def body(t, carry):
        base = base0 + t * CB
        pltpu.sync_copy(idx_i_hbm.at[pl.ds(base, CB)], idx_i_v)
        pltpu.sync_copy(idx_j_hbm.at[pl.ds(base, CB)], idx_j_v)
        ci = pltpu.async_copy(table_hbm.at[idx_i_v], rows_i, sem_i)
        cj = pltpu.async_copy(table_hbm.at[idx_j_v], rows_j, sem_j)
        ci.wait()
        cj.wait()
        pltpu.sync_copy(rows_i, out_i_hbm.at[pl.ds(base, CB)])
        pltpu.sync_copy(rows_j, out_j_hbm.at[pl.ds(base, CB)])
        return carry

    lax.fori_loop(0, ITERS, body, 0)


def _sc_gather(node_feats, idx_i, idx_j):
    mesh = plsc.VectorSubcoreMesh(core_axis_name="c", subcore_axis_name="s")
    k = pl.kernel(
        _gather_body,
        out_type=[
            jax.ShapeDtypeStruct((E, D), jnp.float32),
            jax.ShapeDtypeStruct((E, D), jnp.float32),
        ],
        mesh=mesh,
        scratch_types=[
            pltpu.VMEM((CB,), jnp.int32),
            pltpu.VMEM((CB,), jnp.int32),
            pltpu.VMEM((CB, D), jnp.float32),
            pltpu.VMEM((CB, D), jnp.float32),
            pltpu.SemaphoreType.DMA,
            pltpu.SemaphoreType.DMA,
        ],
    )
    return k(idx_i, idx_j, node_feats)


def _pass1_body(gi_ref, gj_ref, ef_ref, w1a_ref, w1b_ref, w1c_ref,
                z_ref, stats_ref):
    step = pl.program_id(0)
    gi = gi_ref[...]
    gj = gj_ref[...]
    s = gi + gj
    d = jnp.abs(gi - gj)
    z = jnp.dot(s, w1a_ref[...], preferred_element_type=jnp.float32)
    z += jnp.dot(d, w1b_ref[...], preferred_element_type=jnp.float32)
    z += jnp.dot(ef_ref[...], w1c_ref[...], preferred_element_type=jnp.float32)
    z_ref[...] = z

    sums = jnp.sum(z, axis=0)
    sqs = jnp.sum(z * z, axis=0)
    acc = jnp.concatenate(
        [sums[None, :], sqs[None, :], jnp.zeros((6, IDIM), jnp.float32)], axis=0)

    @pl.when(step == 0)
    def _():
        stats_ref[...] = jnp.zeros_like(stats_ref)

    stats_ref[...] += acc


def _pass2_body(z_ref, stats_ref, gamma_ref, beta_ref, w2t_ref, b2_ref,
                out_ref):
    mean = stats_ref[0:1, :] / E
    var = stats_ref[1:2, :] / E - mean * mean
    a = gamma_ref[...] * lax.rsqrt(var + 1e-5)
    c = beta_ref[...] - mean * a
    y = jnp.maximum(z_ref[...] * a + c, 0.0)
    out_ref[...] = (
        jnp.dot(y, w2t_ref[...], preferred_element_type=jnp.float32)
        + b2_ref[...])


def kernel(node_feats, edge_feats, edge_index, W1, b1, gamma, beta, W2, b2):
    del b1
    idx_i = edge_index[0]
    idx_j = edge_index[1]
    gi, gj = _sc_gather(node_feats, idx_i, idx_j)

    w1a = W1[:, :D].T
    w1b = W1[:, D:2 * D].T
    w1c = W1[:, 2 * D:].T

    z, stats = pl.pallas_call(
        _pass1_body,
        grid=(NBLK,),
        in_specs=[
            pl.BlockSpec((BE, D), lambda i: (i, 0)),
            pl.BlockSpec((BE, D), lambda i: (i, 0)),
            pl.BlockSpec((BE, DE), lambda i: (i, 0)),
            pl.BlockSpec((D, IDIM), lambda i: (0, 0)),
            pl.BlockSpec((D, IDIM), lambda i: (0, 0)),
            pl.BlockSpec((DE, IDIM), lambda i: (0, 0)),
        ],
        out_specs=[
            pl.BlockSpec((BE, IDIM), lambda i: (i, 0)),
            pl.BlockSpec((8, IDIM), lambda i: (0, 0)),
        ],
        out_shape=[
            jax.ShapeDtypeStruct((E, IDIM), jnp.float32),
            jax.ShapeDtypeStruct((8, IDIM), jnp.float32),
        ],
    )(gi, gj, edge_feats, w1a, w1b, w1c)

    out = pl.pallas_call(
        _pass2_body,
        grid=(NBLK,),
        in_specs=[
            pl.BlockSpec((BE, IDIM), lambda i: (i, 0)),
            pl.BlockSpec((8, IDIM), lambda i: (0, 0)),
            pl.BlockSpec((1, IDIM), lambda i: (0, 0)),
            pl.BlockSpec((1, IDIM), lambda i: (0, 0)),
            pl.BlockSpec((IDIM, DE), lambda i: (0, 0)),
            pl.BlockSpec((1, DE), lambda i: (0, 0)),
        ],
        out_specs=pl.BlockSpec((BE, DE), lambda i: (i, 0)),
        out_shape=jax.ShapeDtypeStruct((E, DE), jnp.float32),
    )(z, stats, gamma.reshape(1, IDIM), beta.reshape(1, IDIM),
      W2.T, b2.reshape(1, DE))
    return out
